# baseline (speedup 1.0000x reference)
import functools

import jax
import jax.numpy as jnp
from jax import lax
from jax.experimental import pallas as pl
from jax.experimental.pallas import tpu as pltpu
from jax.experimental.pallas import tpu_sc as plsc

B, S, D, E, K, DO = 2, 2048, 1024, 8, 2, 1024
N = B * S
NA = N * K
TB = 256
NBLK = NA // TB + E
CAP = NBLK * TB
NBLK_PAD = 48
NW = 32
AP = NA // NW
TPW = N // NW
DW = D // 2
GSZ = 64
GR = AP // GSZ
CSZ = 32
CGR = AP // CSZ

RT = 1024

_mesh = plsc.VectorSubcoreMesh(core_axis_name="c", subcore_axis_name="s",
                               num_cores=2, num_subcores=16)


def _router_body(x_ref, wr_ref, eid_ref, gate_ref):
    logits = jnp.dot(x_ref[...], wr_ref[...],
                     preferred_element_type=jnp.float32)
    iota = lax.broadcasted_iota(jnp.int32, (RT, E), 1)
    m1 = jnp.max(logits, axis=-1, keepdims=True)
    a1 = jnp.argmax(logits, axis=-1)[:, None]
    masked = jnp.where(iota == a1, -jnp.inf, logits)
    m2 = jnp.max(masked, axis=-1, keepdims=True)
    a2 = jnp.argmax(masked, axis=-1)[:, None]
    z = jnp.exp(m2 - m1)
    w1 = 1.0 / (1.0 + z)
    w2 = z / (1.0 + z)
    eid_ref[...] = jnp.concatenate([a1, a2], axis=1)
    gate_ref[...] = jnp.concatenate([w1, w2], axis=1)


def _router(x2, Wr):
    return pl.pallas_call(
        _router_body,
        grid=(N // RT,),
        in_specs=[
            pl.BlockSpec((RT, D), lambda t: (t, 0)),
            pl.BlockSpec((D, E), lambda t: (0, 0)),
        ],
        out_specs=[
            pl.BlockSpec((RT, K), lambda t: (t, 0)),
            pl.BlockSpec((RT, K), lambda t: (t, 0)),
        ],
        out_shape=[
            jax.ShapeDtypeStruct((N, K), jnp.int32),
            jax.ShapeDtypeStruct((N, K), jnp.float32),
        ],
    )(x2, Wr)


@functools.partial(
    pl.kernel,
    out_type=jax.ShapeDtypeStruct((NW, 16), jnp.int32),
    mesh=_mesh,
    scratch_types=[
        pltpu.VMEM((AP,), jnp.int32),
        pltpu.VMEM((16,), jnp.int32),
    ],
    compiler_params=pltpu.CompilerParams(needs_layout_passes=False),
)
def _hist_kernel(eid_hbm, hist_hbm, eid_v, cnt_v):
    wid = lax.axis_index("s") * 2 + lax.axis_index("c")
    base = wid * AP
    pltpu.sync_copy(eid_hbm.at[pl.ds(base, AP)], eid_v)
    iota = lax.iota(jnp.int32, 16)
    cnt = jnp.zeros((16,), jnp.int32)
    for v in range(AP // 16):
        ev = eid_v[pl.ds(v * 16, 16)]
        for e in range(E):
            pc = jnp.sum(jnp.where(ev == e, 1, 0))
            cnt = cnt + jnp.where(iota == e, pc, 0)
    cnt_v[...] = cnt
    pltpu.sync_copy(cnt_v, hist_hbm.at[wid])


@functools.partial(
    pl.kernel,
    out_type=[
        jax.ShapeDtypeStruct((CAP, DW), jnp.int32),
        jax.ShapeDtypeStruct((CAP,), jnp.float32),
        jax.ShapeDtypeStruct((NA,), jnp.int32),
        jax.ShapeDtypeStruct((NBLK_PAD,), jnp.int32),
        jax.ShapeDtypeStruct((NBLK_PAD,), jnp.int32),
    ],
    mesh=_mesh,
    scratch_types=[
        pltpu.VMEM((AP,), jnp.int32),
        pltpu.VMEM((AP,), jnp.float32),
        pltpu.VMEM((NW, 16), jnp.int32),
        pltpu.VMEM((GR, GSZ), jnp.int32),
        pltpu.VMEM((GR, GSZ), jnp.int32),
        pltpu.VMEM((GSZ, DW), jnp.int32),
        pltpu.VMEM((2 * TPW,), jnp.int32),
        pltpu.VMEM((NBLK_PAD,), jnp.int32),
        pltpu.VMEM((NBLK_PAD,), jnp.int32),
        pltpu.SemaphoreType.DMA,
        pltpu.SemaphoreType.DMA,
    ],
    compiler_params=pltpu.CompilerParams(needs_layout_passes=False),
)
def _sort_kernel(eid_hbm, gate_hbm, xbf_hbm, hist_hbm,
                 xs_hbm, gs_hbm, inv_hbm, bexp_hbm, bsrc_hbm,
                 eid_v, gate_v, hist_v, tsrc_v, dst_v, rows_v, inv_v,
                 m1_v, m2_v, sem, sem2):
    wid = lax.axis_index("s") * 2 + lax.axis_index("c")
    base = wid * AP
    tbase = wid * TPW
    pltpu.sync_copy(eid_hbm.at[pl.ds(base, AP)], eid_v)
    pltpu.sync_copy(gate_hbm.at[pl.ds(base, AP)], gate_v)
    pltpu.sync_copy(hist_hbm, hist_v)
    iota = lax.iota(jnp.int32, 16)

    tot = jnp.zeros((16,), jnp.int32)
    pre = jnp.zeros((16,), jnp.int32)
    for w in range(NW):
        row = hist_v[w]
        tot = tot + row
        pre = pre + jnp.where(jnp.full((16,), w, jnp.int32) < wid, row, 0)
    nblk = (tot + (TB - 1)) // TB
    cum = plsc.cumsum(nblk)
    blk_start = cum - nblk
    slot0 = blk_start * TB + pre
    bases = [jnp.sum(jnp.where(iota == e, slot0, 0)) for e in range(E)]

    for c in range(AP // 16):
        ev = eid_v[pl.ds(c * 16, 16)]
        dest = jnp.zeros((16,), jnp.int32)
        for e in range(E):
            m = ev == e
            mi = jnp.where(m, 1, 0)
            rank = plsc.cumsum(mi)
            dest = jnp.where(m, bases[e] + rank - 1, dest)
            bases[e] = bases[e] + jnp.sum(mi)
        g, o = c // (GSZ // 16), (c % (GSZ // 16)) * 16
        dst_v[g, pl.ds(o, 16)] = dest
        tsrc_v[g, pl.ds(o, 16)] = (base + c * 16 + iota) >> 1
        lt = (c * 16 + iota) >> 1
        kk = (c * 16 + iota) & 1
        plsc.store_scatter(inv_v, [kk * TPW + lt], dest)

    pltpu.sync_copy(inv_v.at[pl.ds(0, TPW)], inv_hbm.at[pl.ds(tbase, TPW)])
    pltpu.sync_copy(inv_v.at[pl.ds(TPW, TPW)],
                    inv_hbm.at[pl.ds(N + tbase, TPW)])

    for g in range(GR):
        pltpu.async_copy(xbf_hbm.at[tsrc_v.at[g]], rows_v, sem).wait()
        pltpu.async_copy(rows_v, xs_hbm.at[dst_v.at[g]], sem2).wait()
        pltpu.async_copy(gate_v.at[pl.ds(g * GSZ, GSZ)],
                         gs_hbm.at[dst_v.at[g]], sem2).wait()

    @pl.when(wid == 0)
    def _meta():
        nact = jnp.sum(jnp.where(iota < E, nblk, 0))
        starts = [jnp.sum(jnp.where(iota == e, blk_start, 0)) for e in range(E)]
        nblk_s = [jnp.sum(jnp.where(iota == e, nblk, 0)) for e in range(E)]
        lastexp = jnp.max(jnp.where((iota < E) & (nblk > 0), iota, 0))
        for r in range(NBLK_PAD // 16):
            bv = iota + r * 16
            expv = jnp.full((16,), 0, jnp.int32)
            for e in range(E):
                expv = jnp.where((bv >= starts[e])
                                 & (bv < starts[e] + nblk_s[e]), e, expv)
            valid = bv < nact
            expv = jnp.where(valid, expv, lastexp)
            srcv = jnp.where(valid, bv, nact - 1)
            m1_v[pl.ds(r * 16, 16)] = expv
            m2_v[pl.ds(r * 16, 16)] = srcv
        pltpu.sync_copy(m1_v, bexp_hbm)
        pltpu.sync_copy(m2_v, bsrc_hbm)


def _gmm_body(bexp_ref, bsrc_ref, xs_ref, we_ref, be_ref, gs_ref, y_ref):
    b = pl.program_id(0)

    @pl.when(bsrc_ref[b] == b)
    def _():
        acc = jnp.dot(xs_ref[...], we_ref[0],
                      preferred_element_type=jnp.float32)
        y_ref[...] = jax.nn.relu(acc + be_ref[0]) * gs_ref[...]


def _gmm(bexp, bsrc, xs_bf, We_bf, be3, gs2):
    grid_spec = pltpu.PrefetchScalarGridSpec(
        num_scalar_prefetch=2,
        grid=(NBLK,),
        in_specs=[
            pl.BlockSpec((TB, D), lambda b, ea, sa: (sa[b], 0)),
            pl.BlockSpec((1, D, DO), lambda b, ea, sa: (ea[b], 0, 0)),
            pl.BlockSpec((1, 1, DO), lambda b, ea, sa: (ea[b], 0, 0)),
            pl.BlockSpec((TB, 1), lambda b, ea, sa: (sa[b], 0)),
        ],
        out_specs=pl.BlockSpec((TB, DO), lambda b, ea, sa: (sa[b], 0)),
    )
    return pl.pallas_call(
        _gmm_body,
        grid_spec=grid_spec,
        out_shape=jax.ShapeDtypeStruct((CAP, DO), jnp.float32),
    )(bexp, bsrc, xs_bf, We_bf, be3, gs2)


CT = 16


@functools.partial(
    pl.kernel,
    out_type=jax.ShapeDtypeStruct((N, DO), jnp.float32),
    mesh=_mesh,
    scratch_types=[
        pltpu.VMEM((AP,), jnp.int32),
        pltpu.VMEM((CT, DO), jnp.float32),
        pltpu.VMEM((CT, DO), jnp.float32),
        pltpu.VMEM((CT, DO), jnp.float32),
        pltpu.SemaphoreType.DMA,
        pltpu.SemaphoreType.DMA,
    ],
    compiler_params=pltpu.CompilerParams(needs_layout_passes=False),
)
def _combine_kernel(y_hbm, inv_hbm, out_hbm, inv_v, ra_v, rb_v, out_v,
                    sem, sem2):
    wid = lax.axis_index("s") * 2 + lax.axis_index("c")
    tbase = wid * TPW
    pltpu.sync_copy(inv_hbm.at[pl.ds(tbase, TPW)], inv_v.at[pl.ds(0, TPW)])
    pltpu.sync_copy(inv_hbm.at[pl.ds(N + tbase, TPW)],
                    inv_v.at[pl.ds(TPW, TPW)])
    for g in range(TPW // CT):
        ca = pltpu.async_copy(y_hbm.at[inv_v.at[pl.ds(g * CT, CT)]],
                              ra_v, sem)
        cb = pltpu.async_copy(y_hbm.at[inv_v.at[pl.ds(TPW + g * CT, CT)]],
                              rb_v, sem2)
        ca.wait()
        cb.wait()
        for i in range(CT):
            def _add(j, _):
                out_v[i, pl.ds(j * 16, 16)] = (ra_v[i, pl.ds(j * 16, 16)]
                                               + rb_v[i, pl.ds(j * 16, 16)])
                return 0
            lax.fori_loop(0, DO // 16, _add, 0)
        pltpu.sync_copy(out_v, out_hbm.at[pl.ds(tbase + g * CT, CT)])


_BISECT = 0

@jax.jit
def _moe(x, Wr, We, be):
    x2 = x.reshape(N, D)
    xbf = lax.bitcast_convert_type(
        x2.astype(jnp.bfloat16).reshape(N, DW, 2), jnp.int32)
    We_bf = We.astype(jnp.bfloat16)
    be3 = be.reshape(E, 1, DO)

    eid, gate = _router(x2, Wr)
    hist = _hist_kernel(eid.reshape(NA))
    if _BISECT == 1:
        return hist.astype(jnp.float32).sum()
    xs, gs, inv, bexp, bsrc = _sort_kernel(
        eid.reshape(NA), gate.reshape(NA), xbf, hist)
    if _BISECT == 2:
        return (xs.astype(jnp.float32).sum() + gs.sum() +
                inv.astype(jnp.float32).sum() + bexp.astype(jnp.float32).sum())
    xs_bf = lax.bitcast_convert_type(xs, jnp.bfloat16).reshape(CAP, D)
    y = _gmm(bexp[:NBLK], bsrc[:NBLK], xs_bf, We_bf, be3,
             gs.reshape(CAP, 1))
    out = _combine_kernel(y, inv)
    return out.reshape(B, S, DO)


def kernel(x, Wr, We, be):
    return _moe(x, Wr, We, be)


# speedup vs baseline: 2.2957x; 2.2957x over previous
import functools

import jax
import jax.numpy as jnp
from jax import lax
from jax.experimental import pallas as pl
from jax.experimental.pallas import tpu as pltpu
from jax.experimental.pallas import tpu_sc as plsc

B, S, D, E, K, DO = 2, 2048, 1024, 8, 2, 1024
N = B * S
NA = N * K
TB = 256
NBLK = NA // TB + E
CAP = NBLK * TB
NBLK_PAD = 48
NW = 32
AP = NA // NW
TPW = N // NW
DW = D // 2
GSZ = 64
GR = AP // GSZ
CSZ = 32
CGR = AP // CSZ

RT = 1024

_mesh = plsc.VectorSubcoreMesh(core_axis_name="c", subcore_axis_name="s",
                               num_cores=2, num_subcores=16)


def _router_body(x_ref, wr_ref, eid_ref, gate_ref):
    logits = jnp.dot(x_ref[...], wr_ref[...],
                     preferred_element_type=jnp.float32)
    iota = lax.broadcasted_iota(jnp.int32, (RT, E), 1)
    m1 = jnp.max(logits, axis=-1, keepdims=True)
    a1 = jnp.argmax(logits, axis=-1)[:, None]
    masked = jnp.where(iota == a1, -jnp.inf, logits)
    m2 = jnp.max(masked, axis=-1, keepdims=True)
    a2 = jnp.argmax(masked, axis=-1)[:, None]
    z = jnp.exp(m2 - m1)
    w1 = 1.0 / (1.0 + z)
    w2 = z / (1.0 + z)
    eid_ref[...] = jnp.concatenate([a1, a2], axis=1)
    gate_ref[...] = jnp.concatenate([w1, w2], axis=1)


def _router(x2, Wr):
    return pl.pallas_call(
        _router_body,
        grid=(N // RT,),
        in_specs=[
            pl.BlockSpec((RT, D), lambda t: (t, 0)),
            pl.BlockSpec((D, E), lambda t: (0, 0)),
        ],
        out_specs=[
            pl.BlockSpec((RT, K), lambda t: (t, 0)),
            pl.BlockSpec((RT, K), lambda t: (t, 0)),
        ],
        out_shape=[
            jax.ShapeDtypeStruct((N, K), jnp.int32),
            jax.ShapeDtypeStruct((N, K), jnp.float32),
        ],
    )(x2, Wr)


@functools.partial(
    pl.kernel,
    out_type=jax.ShapeDtypeStruct((NW, 16), jnp.int32),
    mesh=_mesh,
    scratch_types=[
        pltpu.VMEM((AP,), jnp.int32),
        pltpu.VMEM((16,), jnp.int32),
    ],
    compiler_params=pltpu.CompilerParams(needs_layout_passes=False),
)
def _hist_kernel(eid_hbm, hist_hbm, eid_v, cnt_v):
    wid = lax.axis_index("s") * 2 + lax.axis_index("c")
    base = wid * AP
    pltpu.sync_copy(eid_hbm.at[pl.ds(base, AP)], eid_v)
    iota = lax.iota(jnp.int32, 16)
    cnt = jnp.zeros((16,), jnp.int32)
    for v in range(AP // 16):
        ev = eid_v[pl.ds(v * 16, 16)]
        for e in range(E):
            pc = jnp.sum(jnp.where(ev == e, 1, 0))
            cnt = cnt + jnp.where(iota == e, pc, 0)
    cnt_v[...] = cnt
    pltpu.sync_copy(cnt_v, hist_hbm.at[wid])


@functools.partial(
    pl.kernel,
    out_type=[
        jax.ShapeDtypeStruct((CAP, D), jnp.float32),
        jax.ShapeDtypeStruct((CAP,), jnp.float32),
        jax.ShapeDtypeStruct((NA,), jnp.int32),
        jax.ShapeDtypeStruct((NBLK_PAD,), jnp.int32),
        jax.ShapeDtypeStruct((NBLK_PAD,), jnp.int32),
    ],
    mesh=_mesh,
    scratch_types=[
        pltpu.VMEM((AP,), jnp.int32),
        pltpu.VMEM((AP,), jnp.float32),
        pltpu.VMEM((NW, 16), jnp.int32),
        pltpu.VMEM((GR, GSZ), jnp.int32),
        pltpu.VMEM((GR, GSZ), jnp.int32),
        pltpu.VMEM((GSZ, D), jnp.float32),
        pltpu.VMEM((2 * TPW,), jnp.int32),
        pltpu.VMEM((NBLK_PAD,), jnp.int32),
        pltpu.VMEM((NBLK_PAD,), jnp.int32),
        pltpu.SemaphoreType.DMA,
        pltpu.SemaphoreType.DMA,
    ],
    compiler_params=pltpu.CompilerParams(needs_layout_passes=False),
)
def _sort_kernel(eid_hbm, gate_hbm, xbf_hbm, hist_hbm,
                 xs_hbm, gs_hbm, inv_hbm, bexp_hbm, bsrc_hbm,
                 eid_v, gate_v, hist_v, tsrc_v, dst_v, rows_v, inv_v,
                 m1_v, m2_v, sem, sem2):
    wid = lax.axis_index("s") * 2 + lax.axis_index("c")
    base = wid * AP
    tbase = wid * TPW
    pltpu.sync_copy(eid_hbm.at[pl.ds(base, AP)], eid_v)
    pltpu.sync_copy(gate_hbm.at[pl.ds(base, AP)], gate_v)
    pltpu.sync_copy(hist_hbm, hist_v)
    iota = lax.iota(jnp.int32, 16)

    tot = jnp.zeros((16,), jnp.int32)
    pre = jnp.zeros((16,), jnp.int32)
    for w in range(NW):
        row = hist_v[w]
        tot = tot + row
        pre = pre + jnp.where(jnp.full((16,), w, jnp.int32) < wid, row, 0)
    nblk = (tot + (TB - 1)) // TB
    cum = plsc.cumsum(nblk)
    blk_start = cum - nblk
    slot0 = blk_start * TB + pre
    bases = [jnp.sum(jnp.where(iota == e, slot0, 0)) for e in range(E)]

    for c in range(AP // 16):
        ev = eid_v[pl.ds(c * 16, 16)]
        dest = jnp.zeros((16,), jnp.int32)
        for e in range(E):
            m = ev == e
            mi = jnp.where(m, 1, 0)
            rank = plsc.cumsum(mi)
            dest = jnp.where(m, bases[e] + rank - 1, dest)
            bases[e] = bases[e] + jnp.sum(mi)
        g, o = c // (GSZ // 16), (c % (GSZ // 16)) * 16
        dst_v[g, pl.ds(o, 16)] = dest
        tsrc_v[g, pl.ds(o, 16)] = (base + c * 16 + iota) >> 1
        lt = (c * 16 + iota) >> 1
        kk = (c * 16 + iota) & 1
        plsc.store_scatter(inv_v, [kk * TPW + lt], dest)

    pltpu.sync_copy(inv_v.at[pl.ds(0, TPW)], inv_hbm.at[pl.ds(tbase, TPW)])
    pltpu.sync_copy(inv_v.at[pl.ds(TPW, TPW)],
                    inv_hbm.at[pl.ds(N + tbase, TPW)])

    for g in range(GR):
        pltpu.async_copy(xbf_hbm.at[tsrc_v.at[g]], rows_v, sem).wait()
        pltpu.async_copy(rows_v, xs_hbm.at[dst_v.at[g]], sem2).wait()
        pltpu.async_copy(gate_v.at[pl.ds(g * GSZ, GSZ)],
                         gs_hbm.at[dst_v.at[g]], sem2).wait()

    @pl.when(wid == 0)
    def _meta():
        nact = jnp.sum(jnp.where(iota < E, nblk, 0))
        starts = [jnp.sum(jnp.where(iota == e, blk_start, 0)) for e in range(E)]
        nblk_s = [jnp.sum(jnp.where(iota == e, nblk, 0)) for e in range(E)]
        lastexp = jnp.max(jnp.where((iota < E) & (nblk > 0), iota, 0))
        for r in range(NBLK_PAD // 16):
            bv = iota + r * 16
            expv = jnp.full((16,), 0, jnp.int32)
            for e in range(E):
                expv = jnp.where((bv >= starts[e])
                                 & (bv < starts[e] + nblk_s[e]), e, expv)
            valid = bv < nact
            expv = jnp.where(valid, expv, lastexp)
            srcv = jnp.where(valid, bv, nact - 1)
            m1_v[pl.ds(r * 16, 16)] = expv
            m2_v[pl.ds(r * 16, 16)] = srcv
        pltpu.sync_copy(m1_v, bexp_hbm)
        pltpu.sync_copy(m2_v, bsrc_hbm)


def _gmm_body(bexp_ref, bsrc_ref, xs_ref, we_ref, be_ref, gs_ref, y_ref):
    b = pl.program_id(0)

    @pl.when(bsrc_ref[b] == b)
    def _():
        acc = jnp.dot(xs_ref[...], we_ref[0],
                      preferred_element_type=jnp.float32)
        y_ref[...] = jax.nn.relu(acc + be_ref[0]) * gs_ref[...]


def _gmm(bexp, bsrc, xs_bf, We_bf, be3, gs2):
    grid_spec = pltpu.PrefetchScalarGridSpec(
        num_scalar_prefetch=2,
        grid=(NBLK,),
        in_specs=[
            pl.BlockSpec((TB, D), lambda b, ea, sa: (sa[b], 0)),
            pl.BlockSpec((1, D, DO), lambda b, ea, sa: (ea[b], 0, 0)),
            pl.BlockSpec((1, 1, DO), lambda b, ea, sa: (ea[b], 0, 0)),
            pl.BlockSpec((TB, 1), lambda b, ea, sa: (sa[b], 0)),
        ],
        out_specs=pl.BlockSpec((TB, DO), lambda b, ea, sa: (sa[b], 0)),
    )
    return pl.pallas_call(
        _gmm_body,
        grid_spec=grid_spec,
        out_shape=jax.ShapeDtypeStruct((CAP, DO), jnp.float32),
    )(bexp, bsrc, xs_bf, We_bf, be3, gs2)


CT = 16


@functools.partial(
    pl.kernel,
    out_type=jax.ShapeDtypeStruct((N, DO), jnp.float32),
    mesh=_mesh,
    scratch_types=[
        pltpu.VMEM((AP,), jnp.int32),
        pltpu.VMEM((CT, DO), jnp.float32),
        pltpu.VMEM((CT, DO), jnp.float32),
        pltpu.VMEM((CT, DO), jnp.float32),
        pltpu.SemaphoreType.DMA,
        pltpu.SemaphoreType.DMA,
    ],
    compiler_params=pltpu.CompilerParams(needs_layout_passes=False),
)
def _combine_kernel(y_hbm, inv_hbm, out_hbm, inv_v, ra_v, rb_v, out_v,
                    sem, sem2):
    wid = lax.axis_index("s") * 2 + lax.axis_index("c")
    tbase = wid * TPW
    pltpu.sync_copy(inv_hbm.at[pl.ds(tbase, TPW)], inv_v.at[pl.ds(0, TPW)])
    pltpu.sync_copy(inv_hbm.at[pl.ds(N + tbase, TPW)],
                    inv_v.at[pl.ds(TPW, TPW)])
    for g in range(TPW // CT):
        ca = pltpu.async_copy(y_hbm.at[inv_v.at[pl.ds(g * CT, CT)]],
                              ra_v, sem)
        cb = pltpu.async_copy(y_hbm.at[inv_v.at[pl.ds(TPW + g * CT, CT)]],
                              rb_v, sem2)
        ca.wait()
        cb.wait()
        for i in range(CT):
            def _add(j, _):
                out_v[i, pl.ds(j * 16, 16)] = (ra_v[i, pl.ds(j * 16, 16)]
                                               + rb_v[i, pl.ds(j * 16, 16)])
                return 0
            lax.fori_loop(0, DO // 16, _add, 0)
        pltpu.sync_copy(out_v, out_hbm.at[pl.ds(tbase + g * CT, CT)])


_BISECT = 0

@jax.jit
def _moe(x, Wr, We, be):
    x2 = x.reshape(N, D)
    be3 = be.reshape(E, 1, DO)

    eid, gate = _router(x2, Wr)
    hist = _hist_kernel(eid.reshape(NA))
    if _BISECT == 1:
        return hist.astype(jnp.float32).sum()
    xs, gs, inv, bexp, bsrc = _sort_kernel(
        eid.reshape(NA), gate.reshape(NA), x2, hist)
    if _BISECT == 2:
        return (xs.astype(jnp.float32).sum() + gs.sum() +
                inv.astype(jnp.float32).sum() + bexp.astype(jnp.float32).sum())
    y = _gmm(bexp[:NBLK], bsrc[:NBLK], xs, We, be3,
             gs.reshape(CAP, 1))
    out = _combine_kernel(y, inv)
    return out.reshape(B, S, DO)


def kernel(x, Wr, We, be):
    return _moe(x, Wr, We, be)


# speedup vs baseline: 2.4190x; 1.0537x over previous
import functools

import jax
import jax.numpy as jnp
from jax import lax
from jax.experimental import pallas as pl
from jax.experimental.pallas import tpu as pltpu
from jax.experimental.pallas import tpu_sc as plsc

B, S, D, E, K, DO = 2, 2048, 1024, 8, 2, 1024
N = B * S
NA = N * K
TB = 256
NBLK = NA // TB + E
CAP = NBLK * TB
NBLK_PAD = 48
NW = 32
AP = NA // NW
TPW = N // NW
DW = D // 2
GSZ = 64
GR = AP // GSZ
PG = 32
CSZ = 32
CGR = AP // CSZ

RT = 1024

_mesh = plsc.VectorSubcoreMesh(core_axis_name="c", subcore_axis_name="s",
                               num_cores=2, num_subcores=16)


def _router_body(x_ref, wr_ref, eid_ref, gate_ref):
    logits = jnp.dot(x_ref[...], wr_ref[...],
                     preferred_element_type=jnp.float32)
    iota = lax.broadcasted_iota(jnp.int32, (RT, E), 1)
    m1 = jnp.max(logits, axis=-1, keepdims=True)
    a1 = jnp.argmax(logits, axis=-1)[:, None]
    masked = jnp.where(iota == a1, -jnp.inf, logits)
    m2 = jnp.max(masked, axis=-1, keepdims=True)
    a2 = jnp.argmax(masked, axis=-1)[:, None]
    z = jnp.exp(m2 - m1)
    w1 = 1.0 / (1.0 + z)
    w2 = z / (1.0 + z)
    eid_ref[...] = jnp.concatenate([a1, a2], axis=1)
    gate_ref[...] = jnp.concatenate([w1, w2], axis=1)


def _router(x2, Wr):
    return pl.pallas_call(
        _router_body,
        grid=(N // RT,),
        in_specs=[
            pl.BlockSpec((RT, D), lambda t: (t, 0)),
            pl.BlockSpec((D, E), lambda t: (0, 0)),
        ],
        out_specs=[
            pl.BlockSpec((RT, K), lambda t: (t, 0)),
            pl.BlockSpec((RT, K), lambda t: (t, 0)),
        ],
        out_shape=[
            jax.ShapeDtypeStruct((N, K), jnp.int32),
            jax.ShapeDtypeStruct((N, K), jnp.float32),
        ],
    )(x2, Wr)


@functools.partial(
    pl.kernel,
    out_type=jax.ShapeDtypeStruct((NW, 16), jnp.int32),
    mesh=_mesh,
    scratch_types=[
        pltpu.VMEM((AP,), jnp.int32),
        pltpu.VMEM((16,), jnp.int32),
    ],
    compiler_params=pltpu.CompilerParams(needs_layout_passes=False),
)
def _hist_kernel(eid_hbm, hist_hbm, eid_v, cnt_v):
    wid = lax.axis_index("s") * 2 + lax.axis_index("c")
    base = wid * AP
    pltpu.sync_copy(eid_hbm.at[pl.ds(base, AP)], eid_v)
    iota = lax.iota(jnp.int32, 16)
    cnt = jnp.zeros((16,), jnp.int32)
    for v in range(AP // 16):
        ev = eid_v[pl.ds(v * 16, 16)]
        for e in range(E):
            pc = jnp.sum(jnp.where(ev == e, 1, 0))
            cnt = cnt + jnp.where(iota == e, pc, 0)
    cnt_v[...] = cnt
    pltpu.sync_copy(cnt_v, hist_hbm.at[wid])


@functools.partial(
    pl.kernel,
    out_type=[
        jax.ShapeDtypeStruct((CAP, D), jnp.float32),
        jax.ShapeDtypeStruct((CAP,), jnp.float32),
        jax.ShapeDtypeStruct((NA,), jnp.int32),
        jax.ShapeDtypeStruct((NBLK_PAD,), jnp.int32),
        jax.ShapeDtypeStruct((NBLK_PAD,), jnp.int32),
    ],
    mesh=_mesh,
    scratch_types=[
        pltpu.VMEM((AP,), jnp.int32),
        pltpu.VMEM((AP,), jnp.float32),
        pltpu.VMEM((NW, 16), jnp.int32),
        pltpu.VMEM((GR, GSZ), jnp.int32),
        pltpu.VMEM((2, PG, D), jnp.float32),
        pltpu.VMEM((2 * TPW,), jnp.int32),
        pltpu.VMEM((2 * TPW // PG, PG), jnp.int32),
        pltpu.VMEM((NBLK_PAD,), jnp.int32),
        pltpu.VMEM((NBLK_PAD,), jnp.int32),
        pltpu.SemaphoreType.DMA,
        pltpu.SemaphoreType.DMA,
        pltpu.SemaphoreType.DMA,
        pltpu.SemaphoreType.DMA,
    ],
    compiler_params=pltpu.CompilerParams(needs_layout_passes=False),
)
def _sort_kernel(eid_hbm, gate_hbm, xbf_hbm, hist_hbm,
                 xs_hbm, gs_hbm, inv_hbm, bexp_hbm, bsrc_hbm,
                 eid_v, gate_v, hist_v, dst_v, rows_v, inv_v, inv2_v,
                 m1_v, m2_v, sga0, sga1, ssc0, ssc1):
    wid = lax.axis_index("s") * 2 + lax.axis_index("c")
    base = wid * AP
    tbase = wid * TPW
    pltpu.sync_copy(eid_hbm.at[pl.ds(base, AP)], eid_v)
    pltpu.sync_copy(gate_hbm.at[pl.ds(base, AP)], gate_v)
    pltpu.sync_copy(hist_hbm, hist_v)
    iota = lax.iota(jnp.int32, 16)

    tot = jnp.zeros((16,), jnp.int32)
    pre = jnp.zeros((16,), jnp.int32)
    for w in range(NW):
        row = hist_v[w]
        tot = tot + row
        pre = pre + jnp.where(jnp.full((16,), w, jnp.int32) < wid, row, 0)
    nblk = (tot + (TB - 1)) // TB
    cum = plsc.cumsum(nblk)
    blk_start = cum - nblk
    slot0 = blk_start * TB + pre
    bases = [jnp.sum(jnp.where(iota == e, slot0, 0)) for e in range(E)]

    for c in range(AP // 16):
        ev = eid_v[pl.ds(c * 16, 16)]
        dest = jnp.zeros((16,), jnp.int32)
        for e in range(E):
            m = ev == e
            mi = jnp.where(m, 1, 0)
            rank = plsc.cumsum(mi)
            dest = jnp.where(m, bases[e] + rank - 1, dest)
            bases[e] = bases[e] + jnp.sum(mi)
        g, o = c // (GSZ // 16), (c % (GSZ // 16)) * 16
        dst_v[g, pl.ds(o, 16)] = dest
        lt = (c * 16 + iota) >> 1
        kk = (c * 16 + iota) & 1
        plsc.store_scatter(inv_v, [kk * TPW + lt], dest)

    pltpu.sync_copy(inv_v.at[pl.ds(0, TPW)], inv_hbm.at[pl.ds(tbase, TPW)])
    pltpu.sync_copy(inv_v.at[pl.ds(TPW, TPW)],
                    inv_hbm.at[pl.ds(N + tbase, TPW)])
    for r in range(2 * TPW // PG):
        for h in range(PG // 16):
            inv2_v[r, pl.ds(h * 16, 16)] = inv_v[pl.ds(r * PG + h * 16, 16)]

    ngrp = TPW // PG
    sga = [sga0, sga1]
    ssc = [ssc0, ssc1]

    def _gather(g, p):
        return pltpu.async_copy(
            xbf_hbm.at[pl.ds(tbase + g * PG, PG)], rows_v.at[p], sga[p])

    ga = [_gather(0, 0), _gather(1, 1)]
    for g in range(ngrp):
        p = g % 2
        ga[p].wait()
        s0 = pltpu.async_copy(rows_v.at[p], xs_hbm.at[inv2_v.at[g]], ssc[p])
        s1 = pltpu.async_copy(rows_v.at[p],
                              xs_hbm.at[inv2_v.at[ngrp + g]], ssc[p])
        s0.wait()
        s1.wait()
        if g + 2 < ngrp:
            ga[p] = _gather(g + 2, p)

    for g in range(GR):
        pltpu.sync_copy(gate_v.at[pl.ds(g * GSZ, GSZ)],
                        gs_hbm.at[dst_v.at[g]])

    @pl.when(wid == 0)
    def _meta():
        nact = jnp.sum(jnp.where(iota < E, nblk, 0))
        starts = [jnp.sum(jnp.where(iota == e, blk_start, 0)) for e in range(E)]
        nblk_s = [jnp.sum(jnp.where(iota == e, nblk, 0)) for e in range(E)]
        lastexp = jnp.max(jnp.where((iota < E) & (nblk > 0), iota, 0))
        for r in range(NBLK_PAD // 16):
            bv = iota + r * 16
            expv = jnp.full((16,), 0, jnp.int32)
            for e in range(E):
                expv = jnp.where((bv >= starts[e])
                                 & (bv < starts[e] + nblk_s[e]), e, expv)
            valid = bv < nact
            expv = jnp.where(valid, expv, lastexp)
            srcv = jnp.where(valid, bv, nact - 1)
            m1_v[pl.ds(r * 16, 16)] = expv
            m2_v[pl.ds(r * 16, 16)] = srcv
        pltpu.sync_copy(m1_v, bexp_hbm)
        pltpu.sync_copy(m2_v, bsrc_hbm)


def _gmm_body(bexp_ref, bsrc_ref, xs_ref, we_ref, be_ref, gs_ref, y_ref):
    b = pl.program_id(0)

    @pl.when(bsrc_ref[b] == b)
    def _():
        acc = jnp.dot(xs_ref[...], we_ref[0],
                      preferred_element_type=jnp.float32)
        y_ref[...] = jax.nn.relu(acc + be_ref[0]) * gs_ref[...]


def _gmm(bexp, bsrc, xs_bf, We_bf, be3, gs2):
    grid_spec = pltpu.PrefetchScalarGridSpec(
        num_scalar_prefetch=2,
        grid=(NBLK,),
        in_specs=[
            pl.BlockSpec((TB, D), lambda b, ea, sa: (sa[b], 0)),
            pl.BlockSpec((1, D, DO), lambda b, ea, sa: (ea[b], 0, 0)),
            pl.BlockSpec((1, 1, DO), lambda b, ea, sa: (ea[b], 0, 0)),
            pl.BlockSpec((TB, 1), lambda b, ea, sa: (sa[b], 0)),
        ],
        out_specs=pl.BlockSpec((TB, DO), lambda b, ea, sa: (sa[b], 0)),
    )
    return pl.pallas_call(
        _gmm_body,
        grid_spec=grid_spec,
        out_shape=jax.ShapeDtypeStruct((CAP, DO), jnp.float32),
    )(bexp, bsrc, xs_bf, We_bf, be3, gs2)


CT = 16


@functools.partial(
    pl.kernel,
    out_type=jax.ShapeDtypeStruct((N, DO), jnp.float32),
    mesh=_mesh,
    scratch_types=[
        pltpu.VMEM((AP,), jnp.int32),
        pltpu.VMEM((2, CT, DO), jnp.float32),
        pltpu.VMEM((2, CT, DO), jnp.float32),
        pltpu.VMEM((CT, DO), jnp.float32),
        pltpu.SemaphoreType.DMA,
        pltpu.SemaphoreType.DMA,
        pltpu.SemaphoreType.DMA,
        pltpu.SemaphoreType.DMA,
    ],
    compiler_params=pltpu.CompilerParams(needs_layout_passes=False),
)
def _combine_kernel(y_hbm, inv_hbm, out_hbm, inv_v, ra_v, rb_v, out_v,
                    sa0, sa1, sb0, sb1):
    wid = lax.axis_index("s") * 2 + lax.axis_index("c")
    tbase = wid * TPW
    pltpu.sync_copy(inv_hbm.at[pl.ds(tbase, TPW)], inv_v.at[pl.ds(0, TPW)])
    pltpu.sync_copy(inv_hbm.at[pl.ds(N + tbase, TPW)],
                    inv_v.at[pl.ds(TPW, TPW)])
    ngrp = TPW // CT
    sa = [sa0, sa1]
    sb = [sb0, sb1]

    def _gathers(g, p):
        ca = pltpu.async_copy(y_hbm.at[inv_v.at[pl.ds(g * CT, CT)]],
                              ra_v.at[p], sa[p])
        cb = pltpu.async_copy(y_hbm.at[inv_v.at[pl.ds(TPW + g * CT, CT)]],
                              rb_v.at[p], sb[p])
        return ca, cb

    pend = [_gathers(0, 0), _gathers(1, 1)]
    for g in range(ngrp):
        p = g % 2
        ca, cb = pend[p]
        ca.wait()
        cb.wait()
        for i in range(CT):
            def _add(j, _):
                out_v[i, pl.ds(j * 16, 16)] = (
                    ra_v[p, i, pl.ds(j * 16, 16)]
                    + rb_v[p, i, pl.ds(j * 16, 16)])
                return 0
            lax.fori_loop(0, DO // 16, _add, 0)
        pltpu.sync_copy(out_v, out_hbm.at[pl.ds(tbase + g * CT, CT)])
        if g + 2 < ngrp:
            pend[p] = _gathers(g + 2, p)


_BISECT = 0

@jax.jit
def _moe(x, Wr, We, be):
    x2 = x.reshape(N, D)
    be3 = be.reshape(E, 1, DO)

    eid, gate = _router(x2, Wr)
    hist = _hist_kernel(eid.reshape(NA))
    if _BISECT == 1:
        return hist.astype(jnp.float32).sum()
    xs, gs, inv, bexp, bsrc = _sort_kernel(
        eid.reshape(NA), gate.reshape(NA), x2, hist)
    if _BISECT == 2:
        return (xs.astype(jnp.float32).sum() + gs.sum() +
                inv.astype(jnp.float32).sum() + bexp.astype(jnp.float32).sum())
    y = _gmm(bexp[:NBLK], bsrc[:NBLK], xs, We, be3,
             gs.reshape(CAP, 1))
    out = _combine_kernel(y, inv)
    return out.reshape(B, S, DO)


def kernel(x, Wr, We, be):
    return _moe(x, Wr, We, be)


# speedup vs baseline: 2.4524x; 1.0138x over previous
import functools

import jax
import jax.numpy as jnp
from jax import lax
from jax.experimental import pallas as pl
from jax.experimental.pallas import tpu as pltpu
from jax.experimental.pallas import tpu_sc as plsc

B, S, D, E, K, DO = 2, 2048, 1024, 8, 2, 1024
N = B * S
NA = N * K
TB = 256
NBLK = NA // TB + E
CAP = NBLK * TB
NBLK_PAD = 48
NW = 32
AP = NA // NW
TPW = N // NW
DW = D // 2
GSZ = 64
GR = AP // GSZ
PG = 32
CSZ = 32
CGR = AP // CSZ

RT = 1024

_mesh = plsc.VectorSubcoreMesh(core_axis_name="c", subcore_axis_name="s",
                               num_cores=2, num_subcores=16)


def _router_body(x_ref, wr_ref, eid_ref, gate_ref):
    logits = jnp.dot(x_ref[...], wr_ref[...],
                     preferred_element_type=jnp.float32)
    iota = lax.broadcasted_iota(jnp.int32, (RT, E), 1)
    m1 = jnp.max(logits, axis=-1, keepdims=True)
    a1 = jnp.argmax(logits, axis=-1)[:, None]
    masked = jnp.where(iota == a1, -jnp.inf, logits)
    m2 = jnp.max(masked, axis=-1, keepdims=True)
    a2 = jnp.argmax(masked, axis=-1)[:, None]
    z = jnp.exp(m2 - m1)
    w1 = 1.0 / (1.0 + z)
    w2 = z / (1.0 + z)
    eid_ref[...] = jnp.concatenate([a1, a2], axis=1)
    gate_ref[...] = jnp.concatenate([w1, w2], axis=1)


def _router(x2, Wr):
    return pl.pallas_call(
        _router_body,
        grid=(N // RT,),
        in_specs=[
            pl.BlockSpec((RT, D), lambda t: (t, 0)),
            pl.BlockSpec((D, E), lambda t: (0, 0)),
        ],
        out_specs=[
            pl.BlockSpec((RT, K), lambda t: (t, 0)),
            pl.BlockSpec((RT, K), lambda t: (t, 0)),
        ],
        out_shape=[
            jax.ShapeDtypeStruct((N, K), jnp.int32),
            jax.ShapeDtypeStruct((N, K), jnp.float32),
        ],
    )(x2, Wr)


@functools.partial(
    pl.kernel,
    out_type=jax.ShapeDtypeStruct((NW, 16), jnp.int32),
    mesh=_mesh,
    scratch_types=[
        pltpu.VMEM((AP,), jnp.int32),
        pltpu.VMEM((16,), jnp.int32),
    ],
    compiler_params=pltpu.CompilerParams(needs_layout_passes=False),
)
def _hist_kernel(eid_hbm, hist_hbm, eid_v, cnt_v):
    wid = lax.axis_index("s") * 2 + lax.axis_index("c")
    base = wid * AP
    pltpu.sync_copy(eid_hbm.at[pl.ds(base, AP)], eid_v)
    iota = lax.iota(jnp.int32, 16)
    cnt = jnp.zeros((16,), jnp.int32)
    for v in range(AP // 16):
        ev = eid_v[pl.ds(v * 16, 16)]
        for e in range(E):
            pc = jnp.sum(jnp.where(ev == e, 1, 0))
            cnt = cnt + jnp.where(iota == e, pc, 0)
    cnt_v[...] = cnt
    pltpu.sync_copy(cnt_v, hist_hbm.at[wid])


@functools.partial(
    pl.kernel,
    out_type=[
        jax.ShapeDtypeStruct((CAP, D), jnp.float32),
        jax.ShapeDtypeStruct((CAP,), jnp.float32),
        jax.ShapeDtypeStruct((NA,), jnp.int32),
        jax.ShapeDtypeStruct((NBLK_PAD,), jnp.int32),
        jax.ShapeDtypeStruct((NBLK_PAD,), jnp.int32),
    ],
    mesh=_mesh,
    scratch_types=[
        pltpu.VMEM((AP,), jnp.int32),
        pltpu.VMEM((AP,), jnp.float32),
        pltpu.VMEM((NW, 16), jnp.int32),
        pltpu.VMEM((GR, GSZ), jnp.int32),
        pltpu.VMEM((2, PG, D), jnp.float32),
        pltpu.VMEM((2 * TPW,), jnp.int32),
        pltpu.VMEM((2 * TPW // PG, PG), jnp.int32),
        pltpu.VMEM((NBLK_PAD,), jnp.int32),
        pltpu.VMEM((NBLK_PAD,), jnp.int32),
        pltpu.SemaphoreType.DMA,
        pltpu.SemaphoreType.DMA,
        pltpu.SemaphoreType.DMA,
        pltpu.SemaphoreType.DMA,
    ],
    compiler_params=pltpu.CompilerParams(needs_layout_passes=False),
)
def _sort_kernel(eid_hbm, gate_hbm, xbf_hbm, hist_hbm,
                 xs_hbm, gs_hbm, inv_hbm, bexp_hbm, bsrc_hbm,
                 eid_v, gate_v, hist_v, dst_v, rows_v, inv_v, inv2_v,
                 m1_v, m2_v, sga0, sga1, ssc0, ssc1):
    wid = lax.axis_index("s") * 2 + lax.axis_index("c")
    base = wid * AP
    tbase = wid * TPW
    pltpu.sync_copy(eid_hbm.at[pl.ds(base, AP)], eid_v)
    pltpu.sync_copy(gate_hbm.at[pl.ds(base, AP)], gate_v)
    pltpu.sync_copy(hist_hbm, hist_v)
    iota = lax.iota(jnp.int32, 16)

    tot = jnp.zeros((16,), jnp.int32)
    pre = jnp.zeros((16,), jnp.int32)
    for w in range(NW):
        row = hist_v[w]
        tot = tot + row
        pre = pre + jnp.where(jnp.full((16,), w, jnp.int32) < wid, row, 0)
    nblk = (tot + (TB - 1)) // TB
    cum = plsc.cumsum(nblk)
    blk_start = cum - nblk
    slot0 = blk_start * TB + pre
    bases = [jnp.sum(jnp.where(iota == e, slot0, 0)) for e in range(E)]

    for c in range(AP // 16):
        ev = eid_v[pl.ds(c * 16, 16)]
        dest = jnp.zeros((16,), jnp.int32)
        for e in range(E):
            m = ev == e
            mi = jnp.where(m, 1, 0)
            rank = plsc.cumsum(mi)
            dest = jnp.where(m, bases[e] + rank - 1, dest)
            bases[e] = bases[e] + jnp.sum(mi)
        g, o = c // (GSZ // 16), (c % (GSZ // 16)) * 16
        dst_v[g, pl.ds(o, 16)] = dest
        lt = (c * 16 + iota) >> 1
        kk = (c * 16 + iota) & 1
        plsc.store_scatter(inv_v, [kk * TPW + lt], dest)

    pltpu.sync_copy(inv_v.at[pl.ds(0, TPW)], inv_hbm.at[pl.ds(tbase, TPW)])
    pltpu.sync_copy(inv_v.at[pl.ds(TPW, TPW)],
                    inv_hbm.at[pl.ds(N + tbase, TPW)])
    for r in range(2 * TPW // PG):
        for h in range(PG // 16):
            inv2_v[r, pl.ds(h * 16, 16)] = inv_v[pl.ds(r * PG + h * 16, 16)]

    ngrp = TPW // PG
    sga = [sga0, sga1]
    ssc = [ssc0, ssc1]

    def _gather(g, p):
        return pltpu.async_copy(
            xbf_hbm.at[pl.ds(tbase + g * PG, PG)], rows_v.at[p], sga[p])

    ga = [_gather(0, 0), _gather(1, 1)]
    scp = [None, None]
    for g in range(ngrp):
        p = g % 2
        ga[p].wait()
        s0 = pltpu.async_copy(rows_v.at[p], xs_hbm.at[inv2_v.at[g]], ssc[p])
        s1 = pltpu.async_copy(rows_v.at[p],
                              xs_hbm.at[inv2_v.at[ngrp + g]], ssc[p])
        scp[p] = (s0, s1)
        op = 1 - p
        if scp[op] is not None and g + 1 < ngrp:
            scp[op][0].wait()
            scp[op][1].wait()
            scp[op] = None
            ga[op] = _gather(g + 1, op)
    for q in range(2):
        if scp[q] is not None:
            scp[q][0].wait()
            scp[q][1].wait()

    for g in range(GR):
        pltpu.sync_copy(gate_v.at[pl.ds(g * GSZ, GSZ)],
                        gs_hbm.at[dst_v.at[g]])

    @pl.when(wid == 0)
    def _meta():
        nact = jnp.sum(jnp.where(iota < E, nblk, 0))
        starts = [jnp.sum(jnp.where(iota == e, blk_start, 0)) for e in range(E)]
        nblk_s = [jnp.sum(jnp.where(iota == e, nblk, 0)) for e in range(E)]
        lastexp = jnp.max(jnp.where((iota < E) & (nblk > 0), iota, 0))
        for r in range(NBLK_PAD // 16):
            bv = iota + r * 16
            expv = jnp.full((16,), 0, jnp.int32)
            for e in range(E):
                expv = jnp.where((bv >= starts[e])
                                 & (bv < starts[e] + nblk_s[e]), e, expv)
            valid = bv < nact
            expv = jnp.where(valid, expv, lastexp)
            srcv = jnp.where(valid, bv, nact - 1)
            m1_v[pl.ds(r * 16, 16)] = expv
            m2_v[pl.ds(r * 16, 16)] = srcv
        pltpu.sync_copy(m1_v, bexp_hbm)
        pltpu.sync_copy(m2_v, bsrc_hbm)


def _gmm_body(bexp_ref, bsrc_ref, xs_ref, we_ref, be_ref, gs_ref, y_ref):
    b = pl.program_id(0)

    @pl.when(bsrc_ref[b] == b)
    def _():
        acc = jnp.dot(xs_ref[...], we_ref[0],
                      preferred_element_type=jnp.float32)
        y_ref[...] = jax.nn.relu(acc + be_ref[0]) * gs_ref[...]


def _gmm(bexp, bsrc, xs_bf, We_bf, be3, gs2):
    grid_spec = pltpu.PrefetchScalarGridSpec(
        num_scalar_prefetch=2,
        grid=(NBLK,),
        in_specs=[
            pl.BlockSpec((TB, D), lambda b, ea, sa: (sa[b], 0)),
            pl.BlockSpec((1, D, DO), lambda b, ea, sa: (ea[b], 0, 0)),
            pl.BlockSpec((1, 1, DO), lambda b, ea, sa: (ea[b], 0, 0)),
            pl.BlockSpec((TB, 1), lambda b, ea, sa: (sa[b], 0)),
        ],
        out_specs=pl.BlockSpec((TB, DO), lambda b, ea, sa: (sa[b], 0)),
    )
    return pl.pallas_call(
        _gmm_body,
        grid_spec=grid_spec,
        out_shape=jax.ShapeDtypeStruct((CAP, DO), jnp.float32),
    )(bexp, bsrc, xs_bf, We_bf, be3, gs2)


CT = 16


@functools.partial(
    pl.kernel,
    out_type=jax.ShapeDtypeStruct((N, DO), jnp.float32),
    mesh=_mesh,
    scratch_types=[
        pltpu.VMEM((AP,), jnp.int32),
        pltpu.VMEM((2, CT, DO), jnp.float32),
        pltpu.VMEM((2, CT, DO), jnp.float32),
        pltpu.VMEM((CT, DO), jnp.float32),
        pltpu.SemaphoreType.DMA,
        pltpu.SemaphoreType.DMA,
        pltpu.SemaphoreType.DMA,
        pltpu.SemaphoreType.DMA,
    ],
    compiler_params=pltpu.CompilerParams(needs_layout_passes=False),
)
def _combine_kernel(y_hbm, inv_hbm, out_hbm, inv_v, ra_v, rb_v, out_v,
                    sa0, sa1, sb0, sb1):
    wid = lax.axis_index("s") * 2 + lax.axis_index("c")
    tbase = wid * TPW
    pltpu.sync_copy(inv_hbm.at[pl.ds(tbase, TPW)], inv_v.at[pl.ds(0, TPW)])
    pltpu.sync_copy(inv_hbm.at[pl.ds(N + tbase, TPW)],
                    inv_v.at[pl.ds(TPW, TPW)])
    ngrp = TPW // CT
    sa = [sa0, sa1]
    sb = [sb0, sb1]

    def _gathers(g, p):
        ca = pltpu.async_copy(y_hbm.at[inv_v.at[pl.ds(g * CT, CT)]],
                              ra_v.at[p], sa[p])
        cb = pltpu.async_copy(y_hbm.at[inv_v.at[pl.ds(TPW + g * CT, CT)]],
                              rb_v.at[p], sb[p])
        return ca, cb

    pend = [_gathers(0, 0), _gathers(1, 1)]
    for g in range(ngrp):
        p = g % 2
        ca, cb = pend[p]
        ca.wait()
        cb.wait()
        for i in range(CT):
            def _add(j, _):
                out_v[i, pl.ds(j * 16, 16)] = (
                    ra_v[p, i, pl.ds(j * 16, 16)]
                    + rb_v[p, i, pl.ds(j * 16, 16)])
                return 0
            lax.fori_loop(0, DO // 16, _add, 0)
        pltpu.sync_copy(out_v, out_hbm.at[pl.ds(tbase + g * CT, CT)])
        if g + 2 < ngrp:
            pend[p] = _gathers(g + 2, p)


_BISECT = 0

@jax.jit
def _moe(x, Wr, We, be):
    x2 = x.reshape(N, D)
    be3 = be.reshape(E, 1, DO)

    eid, gate = _router(x2, Wr)
    hist = _hist_kernel(eid.reshape(NA))
    if _BISECT == 1:
        return hist.astype(jnp.float32).sum()
    xs, gs, inv, bexp, bsrc = _sort_kernel(
        eid.reshape(NA), gate.reshape(NA), x2, hist)
    if _BISECT == 2:
        return (xs.astype(jnp.float32).sum() + gs.sum() +
                inv.astype(jnp.float32).sum() + bexp.astype(jnp.float32).sum())
    y = _gmm(bexp[:NBLK], bsrc[:NBLK], xs, We, be3,
             gs.reshape(CAP, 1))
    out = _combine_kernel(y, inv)
    return out.reshape(B, S, DO)


def kernel(x, Wr, We, be):
    return _moe(x, Wr, We, be)


# speedup vs baseline: 3.0666x; 1.2505x over previous
import functools

import jax
import jax.numpy as jnp
from jax import lax
from jax.experimental import pallas as pl
from jax.experimental.pallas import tpu as pltpu
from jax.experimental.pallas import tpu_sc as plsc

B, S, D, E, K, DO = 2, 2048, 1024, 8, 2, 1024
N = B * S
NA = N * K
TB = 256
NBLK = NA // TB + E
CAP = NBLK * TB
NBLK_PAD = 48
NW = 32
AP = NA // NW
TPW = N // NW
DW = D // 2
GSZ = 64
GR = AP // GSZ
PG = 32
CSZ = 32
CGR = AP // CSZ

RT = 1024

_mesh = plsc.VectorSubcoreMesh(core_axis_name="c", subcore_axis_name="s",
                               num_cores=2, num_subcores=16)


def _router_body(x_ref, wr_ref, eid_ref, gate_ref):
    logits = jnp.dot(x_ref[...], wr_ref[...],
                     preferred_element_type=jnp.float32)
    iota = lax.broadcasted_iota(jnp.int32, (RT, E), 1)
    m1 = jnp.max(logits, axis=-1, keepdims=True)
    a1 = jnp.argmax(logits, axis=-1)[:, None]
    masked = jnp.where(iota == a1, -jnp.inf, logits)
    m2 = jnp.max(masked, axis=-1, keepdims=True)
    a2 = jnp.argmax(masked, axis=-1)[:, None]
    z = jnp.exp(m2 - m1)
    w1 = 1.0 / (1.0 + z)
    w2 = z / (1.0 + z)
    eid_ref[...] = jnp.concatenate([a1, a2], axis=1)
    gate_ref[...] = jnp.concatenate([w1, w2], axis=1)


def _router(x2, Wr):
    return pl.pallas_call(
        _router_body,
        grid=(N // RT,),
        in_specs=[
            pl.BlockSpec((RT, D), lambda t: (t, 0)),
            pl.BlockSpec((D, E), lambda t: (0, 0)),
        ],
        out_specs=[
            pl.BlockSpec((RT, K), lambda t: (t, 0)),
            pl.BlockSpec((RT, K), lambda t: (t, 0)),
        ],
        out_shape=[
            jax.ShapeDtypeStruct((N, K), jnp.int32),
            jax.ShapeDtypeStruct((N, K), jnp.float32),
        ],
    )(x2, Wr)


@functools.partial(
    pl.kernel,
    out_type=jax.ShapeDtypeStruct((NW, 16), jnp.int32),
    mesh=_mesh,
    scratch_types=[
        pltpu.VMEM((AP,), jnp.int32),
        pltpu.VMEM((16,), jnp.int32),
    ],
    compiler_params=pltpu.CompilerParams(needs_layout_passes=False),
)
def _hist_kernel(eid_hbm, hist_hbm, eid_v, cnt_v):
    wid = lax.axis_index("s") * 2 + lax.axis_index("c")
    base = wid * AP
    pltpu.sync_copy(eid_hbm.at[pl.ds(base, AP)], eid_v)
    iota = lax.iota(jnp.int32, 16)
    cnt = jnp.zeros((16,), jnp.int32)
    for v in range(AP // 16):
        ev = eid_v[pl.ds(v * 16, 16)]
        for e in range(E):
            pc = jnp.sum(jnp.where(ev == e, 1, 0))
            cnt = cnt + jnp.where(iota == e, pc, 0)
    cnt_v[...] = cnt
    pltpu.sync_copy(cnt_v, hist_hbm.at[wid])


@functools.partial(
    pl.kernel,
    out_type=[
        jax.ShapeDtypeStruct((CAP, D), jnp.float32),
        jax.ShapeDtypeStruct((NA,), jnp.int32),
        jax.ShapeDtypeStruct((NBLK_PAD,), jnp.int32),
        jax.ShapeDtypeStruct((NBLK_PAD,), jnp.int32),
    ],
    mesh=_mesh,
    scratch_types=[
        pltpu.VMEM((AP,), jnp.int32),
        pltpu.VMEM((NW, 16), jnp.int32),
        pltpu.VMEM((2, PG, D), jnp.float32),
        pltpu.VMEM((2 * TPW,), jnp.int32),
        pltpu.VMEM((2 * TPW // PG, PG), jnp.int32),
        pltpu.VMEM((NBLK_PAD,), jnp.int32),
        pltpu.VMEM((NBLK_PAD,), jnp.int32),
        pltpu.SemaphoreType.DMA,
        pltpu.SemaphoreType.DMA,
        pltpu.SemaphoreType.DMA,
        pltpu.SemaphoreType.DMA,
    ],
    compiler_params=pltpu.CompilerParams(needs_layout_passes=False),
)
def _sort_kernel(eid_hbm, xbf_hbm, hist_hbm,
                 xs_hbm, inv_hbm, bexp_hbm, bsrc_hbm,
                 eid_v, hist_v, rows_v, inv_v, inv2_v,
                 m1_v, m2_v, sga0, sga1, ssc0, ssc1):
    wid = lax.axis_index("s") * 2 + lax.axis_index("c")
    base = wid * AP
    tbase = wid * TPW
    pltpu.sync_copy(eid_hbm.at[pl.ds(base, AP)], eid_v)
    pltpu.sync_copy(hist_hbm, hist_v)
    iota = lax.iota(jnp.int32, 16)

    tot = jnp.zeros((16,), jnp.int32)
    pre = jnp.zeros((16,), jnp.int32)
    for w in range(NW):
        row = hist_v[w]
        tot = tot + row
        pre = pre + jnp.where(jnp.full((16,), w, jnp.int32) < wid, row, 0)
    nblk = (tot + (TB - 1)) // TB
    cum = plsc.cumsum(nblk)
    blk_start = cum - nblk
    slot0 = blk_start * TB + pre
    bases = [jnp.sum(jnp.where(iota == e, slot0, 0)) for e in range(E)]

    for c in range(AP // 16):
        ev = eid_v[pl.ds(c * 16, 16)]
        dest = jnp.zeros((16,), jnp.int32)
        for e in range(E):
            m = ev == e
            mi = jnp.where(m, 1, 0)
            rank = plsc.cumsum(mi)
            dest = jnp.where(m, bases[e] + rank - 1, dest)
            bases[e] = bases[e] + jnp.sum(mi)
        lt = (c * 16 + iota) >> 1
        kk = (c * 16 + iota) & 1
        plsc.store_scatter(inv_v, [kk * TPW + lt], dest)

    pltpu.sync_copy(inv_v.at[pl.ds(0, TPW)], inv_hbm.at[pl.ds(tbase, TPW)])
    pltpu.sync_copy(inv_v.at[pl.ds(TPW, TPW)],
                    inv_hbm.at[pl.ds(N + tbase, TPW)])
    for r in range(2 * TPW // PG):
        for h in range(PG // 16):
            inv2_v[r, pl.ds(h * 16, 16)] = inv_v[pl.ds(r * PG + h * 16, 16)]

    ngrp = TPW // PG
    sga = [sga0, sga1]
    ssc = [ssc0, ssc1]

    def _gather(g, p):
        return pltpu.async_copy(
            xbf_hbm.at[pl.ds(tbase + g * PG, PG)], rows_v.at[p], sga[p])

    ga = [_gather(0, 0), _gather(1, 1)]
    scp = [None, None]
    for g in range(ngrp):
        p = g % 2
        ga[p].wait()
        s0 = pltpu.async_copy(rows_v.at[p], xs_hbm.at[inv2_v.at[g]], ssc[p])
        s1 = pltpu.async_copy(rows_v.at[p],
                              xs_hbm.at[inv2_v.at[ngrp + g]], ssc[p])
        scp[p] = (s0, s1)
        op = 1 - p
        if scp[op] is not None and g + 1 < ngrp:
            scp[op][0].wait()
            scp[op][1].wait()
            scp[op] = None
            ga[op] = _gather(g + 1, op)
    for q in range(2):
        if scp[q] is not None:
            scp[q][0].wait()
            scp[q][1].wait()

    @pl.when(wid == 0)
    def _meta():
        nact = jnp.sum(jnp.where(iota < E, nblk, 0))
        starts = [jnp.sum(jnp.where(iota == e, blk_start, 0)) for e in range(E)]
        nblk_s = [jnp.sum(jnp.where(iota == e, nblk, 0)) for e in range(E)]
        lastexp = jnp.max(jnp.where((iota < E) & (nblk > 0), iota, 0))
        for r in range(NBLK_PAD // 16):
            bv = iota + r * 16
            expv = jnp.full((16,), 0, jnp.int32)
            for e in range(E):
                expv = jnp.where((bv >= starts[e])
                                 & (bv < starts[e] + nblk_s[e]), e, expv)
            valid = bv < nact
            expv = jnp.where(valid, expv, lastexp)
            srcv = jnp.where(valid, bv, nact - 1)
            m1_v[pl.ds(r * 16, 16)] = expv
            m2_v[pl.ds(r * 16, 16)] = srcv
        pltpu.sync_copy(m1_v, bexp_hbm)
        pltpu.sync_copy(m2_v, bsrc_hbm)


def _gmm_body(bexp_ref, bsrc_ref, xs_ref, we_ref, be_ref, y_ref):
    b = pl.program_id(0)

    @pl.when(bsrc_ref[b] == b)
    def _():
        acc = jnp.dot(xs_ref[...], we_ref[0],
                      preferred_element_type=jnp.float32)
        y_ref[...] = jax.nn.relu(acc + be_ref[0])


def _gmm(bexp, bsrc, xs_bf, We_bf, be3):
    grid_spec = pltpu.PrefetchScalarGridSpec(
        num_scalar_prefetch=2,
        grid=(NBLK,),
        in_specs=[
            pl.BlockSpec((TB, D), lambda b, ea, sa: (sa[b], 0)),
            pl.BlockSpec((1, D, DO), lambda b, ea, sa: (ea[b], 0, 0)),
            pl.BlockSpec((1, 1, DO), lambda b, ea, sa: (ea[b], 0, 0)),
        ],
        out_specs=pl.BlockSpec((TB, DO), lambda b, ea, sa: (sa[b], 0)),
    )
    return pl.pallas_call(
        _gmm_body,
        grid_spec=grid_spec,
        out_shape=jax.ShapeDtypeStruct((CAP, DO), jnp.float32),
    )(bexp, bsrc, xs_bf, We_bf, be3)


CT = 16


@functools.partial(
    pl.kernel,
    out_type=jax.ShapeDtypeStruct((N, DO), jnp.float32),
    mesh=_mesh,
    scratch_types=[
        pltpu.VMEM((AP,), jnp.int32),
        pltpu.VMEM((AP,), jnp.float32),
        pltpu.VMEM((2, CT, DO), jnp.float32),
        pltpu.VMEM((2, CT, DO), jnp.float32),
        pltpu.VMEM((CT, DO), jnp.float32),
        pltpu.SemaphoreType.DMA,
        pltpu.SemaphoreType.DMA,
        pltpu.SemaphoreType.DMA,
        pltpu.SemaphoreType.DMA,
    ],
    compiler_params=pltpu.CompilerParams(needs_layout_passes=False),
)
def _combine_kernel(y_hbm, inv_hbm, gate_hbm, out_hbm, inv_v, gate_v,
                    ra_v, rb_v, out_v, sa0, sa1, sb0, sb1):
    wid = lax.axis_index("s") * 2 + lax.axis_index("c")
    tbase = wid * TPW
    pltpu.sync_copy(inv_hbm.at[pl.ds(tbase, TPW)], inv_v.at[pl.ds(0, TPW)])
    pltpu.sync_copy(inv_hbm.at[pl.ds(N + tbase, TPW)],
                    inv_v.at[pl.ds(TPW, TPW)])
    pltpu.sync_copy(gate_hbm.at[pl.ds(wid * AP, AP)], gate_v)
    ngrp = TPW // CT
    sa = [sa0, sa1]
    sb = [sb0, sb1]

    def _gathers(g, p):
        ca = pltpu.async_copy(y_hbm.at[inv_v.at[pl.ds(g * CT, CT)]],
                              ra_v.at[p], sa[p])
        cb = pltpu.async_copy(y_hbm.at[inv_v.at[pl.ds(TPW + g * CT, CT)]],
                              rb_v.at[p], sb[p])
        return ca, cb

    pend = [_gathers(0, 0), _gathers(1, 1)]
    for g in range(ngrp):
        p = g % 2
        ca, cb = pend[p]
        ca.wait()
        cb.wait()
        gva = gate_v[pl.ds(g * 2 * CT, 16)]
        gvb = gate_v[pl.ds(g * 2 * CT + 16, 16)]
        for i in range(CT):
            gv = gva if i < 8 else gvb
            g0 = gv[2 * (i % 8)]
            g1 = gv[2 * (i % 8) + 1]

            def _add(j, _):
                out_v[i, pl.ds(j * 16, 16)] = (
                    g0 * ra_v[p, i, pl.ds(j * 16, 16)]
                    + g1 * rb_v[p, i, pl.ds(j * 16, 16)])
                return 0
            lax.fori_loop(0, DO // 16, _add, 0)
        pltpu.sync_copy(out_v, out_hbm.at[pl.ds(tbase + g * CT, CT)])
        if g + 2 < ngrp:
            pend[p] = _gathers(g + 2, p)


@jax.jit
def _moe(x, Wr, We, be):
    x2 = x.reshape(N, D)
    be3 = be.reshape(E, 1, DO)

    eid, gate = _router(x2, Wr)
    hist = _hist_kernel(eid.reshape(NA))
    xs, inv, bexp, bsrc = _sort_kernel(eid.reshape(NA), x2, hist)
    y = _gmm(bexp[:NBLK], bsrc[:NBLK], xs, We, be3)
    out = _combine_kernel(y, inv, gate.reshape(NA))
    return out.reshape(B, S, DO)


def kernel(x, Wr, We, be):
    return _moe(x, Wr, We, be)
